# baseline (speedup 1.0000x reference)
import jax
import jax.numpy as jnp
import numpy as np
from jax import lax
from jax.experimental import pallas as pl
from jax.experimental.pallas import tpu as pltpu
from jax.experimental.pallas import tpu_sc as plsc

N = 10000
E = 320000
D = 128
EMB = 16
H = 64

_z = np.linspace(-10.0, 10.0, 200001)
_pdf = np.exp(-0.5 * _z ** 2) / np.sqrt(2.0 * np.pi)
_a = np.logaddexp(0.0, _z) - np.log(2.0)
_SSP = float(1.0 / np.sqrt(np.trapz(_a ** 2 * _pdf, _z)))
_LOG2 = float(np.log(2.0))

G = 2
EG = E // G

_NC = 2
_NS = 16
_NW = _NC * _NS
CH = 80
NCHUNK = EG // CH
ITERS = (NCHUNK + _NW - 1) // _NW
N_PAD = 10240
ROWS_PER_SUB = N_PAD // _NS

BLK = 3200
BPG = EG // BLK


def _ssp(x):
    sp = jnp.maximum(x, 0.0) + jnp.log(1.0 + jnp.exp(-jnp.abs(x)))
    return (sp - _LOG2) * _SSP


def _coef_body(embt_ref, attr_ref, w1_ref, w2_ref, w3_ref, o_ref):
    h = _ssp(lax.dot_general(embt_ref[...], w1_ref[...],
                             (((0,), (0,)), ((), ())),
                             preferred_element_type=jnp.float32))
    h = _ssp(jnp.dot(h, w2_ref[...], preferred_element_type=jnp.float32))
    w = jnp.dot(h, w3_ref[...], preferred_element_type=jnp.float32)
    a = jnp.transpose(attr_ref[...])
    o_ref[...] = w * a


def _edge_coefficients(g, edge_embedding_t, edge_attr_t, W1a, W2a, W3a):
    return pl.pallas_call(
        _coef_body,
        grid=(BPG,),
        in_specs=[
            pl.BlockSpec((EMB, BLK), lambda i: (0, i + g * BPG)),
            pl.BlockSpec((1, BLK), lambda i: (0, i + g * BPG)),
            pl.BlockSpec((EMB, H), lambda i: (0, 0)),
            pl.BlockSpec((H, H), lambda i: (0, 0)),
            pl.BlockSpec((H, D), lambda i: (0, 0)),
        ],
        out_specs=pl.BlockSpec((BLK, D), lambda i: (i, 0)),
        out_shape=jax.ShapeDtypeStruct((EG, D), jnp.float32),
    )(edge_embedding_t, edge_attr_t, W1a, W2a, W3a)


def _sc_body(x_hbm, p_hbm, src_hbm, dst_hbm, out_hbm,
             src_v, dst_v, sdst_v, rows_v, p_v, acc_sh,
             s_src, s_dst, s_g, s_p):
    cid = lax.axis_index("c")
    sid = lax.axis_index("s")
    wid = sid * _NC + cid

    def _base(j):
        return (wid + j * _NW) * CH

    def _valid(j):
        return (wid + j * _NW) < NCHUNK

    def _issue_idx(j, b):
        pltpu.async_copy(src_hbm.at[pl.ds(_base(j), CH)], src_v.at[b], s_src.at[b])
        pltpu.async_copy(dst_hbm.at[pl.ds(_base(j), CH)], dst_v.at[b], s_dst.at[b])

    def _wait_idx(j, b):
        pltpu.make_async_copy(src_hbm.at[pl.ds(_base(j), CH)], src_v.at[b], s_src.at[b]).wait()
        pltpu.make_async_copy(dst_hbm.at[pl.ds(_base(j), CH)], dst_v.at[b], s_dst.at[b]).wait()

    def _issue_data(j, b):
        pltpu.async_copy(x_hbm.at[src_v.at[b]], rows_v.at[b], s_g.at[b])
        pltpu.async_copy(p_hbm.at[pl.ds(_base(j), CH)], p_v.at[b], s_p.at[b])

    def _wait_data(j, b):
        pltpu.make_async_copy(x_hbm.at[src_v.at[b]], rows_v.at[b], s_g.at[b]).wait()
        pltpu.make_async_copy(p_hbm.at[pl.ds(_base(j), CH)], p_v.at[b], s_p.at[b]).wait()

    def _zrow(i, carry):
        for k in range(D // 16):
            rows_v[0, i, pl.ds(k * 16, 16)] = jnp.zeros((16,), jnp.float32)
        return carry
    lax.fori_loop(0, CH, _zrow, 0)
    for t in range(ROWS_PER_SUB // CH):
        pltpu.sync_copy(rows_v.at[0],
                        acc_sh.at[pl.ds(sid * ROWS_PER_SUB + t * CH, CH)])
    plsc.subcore_barrier()

    pltpu.sync_copy(src_hbm.at[pl.ds(_base(0), CH)], src_v.at[0])
    pltpu.sync_copy(dst_hbm.at[pl.ds(_base(0), CH)], dst_v.at[0])
    _issue_data(0, 0)
    _issue_idx(1, 1)

    def _outer(g, carry):
        for b in (0, 1):
            j = g * 2 + b
            b2 = 1 - b

            @pl.when(_valid(j + 1))
            def _():
                _wait_idx(j + 1, b2)
                _issue_data(j + 1, b2)

            @pl.when(_valid(j))
            def _():
                _wait_data(j, b)
                for k in range(CH // 16):
                    s = pl.ds(k * 16, 16)
                    sdst_v[b, s] = dst_v[b, s]

                @pl.when(_valid(j + 2))
                def _():
                    _issue_idx(j + 2, b)

                def _mul(i, c2):
                    for k in range(D // 16):
                        s = pl.ds(k * 16, 16)
                        p_v[b, i, s] = p_v[b, i, s] * rows_v[b, i, s]
                    return c2
                lax.fori_loop(0, CH, _mul, 0)
                pltpu.sync_copy(p_v.at[b], acc_sh.at[sdst_v.at[b]], add=True)
        return carry
    lax.fori_loop(0, (ITERS + 1) // 2, _outer, 0)
    plsc.subcore_barrier()

    for t in range(ROWS_PER_SUB // CH):
        r = sid * ROWS_PER_SUB + t * CH
        pltpu.sync_copy(acc_sh.at[pl.ds(r, CH)], rows_v.at[0])
        pltpu.sync_copy(rows_v.at[0], out_hbm.at[cid, pl.ds(r, CH)])


def _scatter_partials(node_features, coef, edge_src, edge_dst):
    mesh = plsc.VectorSubcoreMesh(core_axis_name="c", subcore_axis_name="s")
    f = pl.kernel(
        _sc_body,
        out_type=jax.ShapeDtypeStruct((_NC, N_PAD, D), jnp.float32),
        mesh=mesh,
        scratch_types=[
            pltpu.VMEM((2, CH), jnp.int32),
            pltpu.VMEM((2, CH), jnp.int32),
            pltpu.VMEM((2, CH), jnp.int32),
            pltpu.VMEM((2, CH, D), jnp.float32),
            pltpu.VMEM((2, CH, D), jnp.float32),
            pltpu.VMEM_SHARED((N_PAD, D), jnp.float32),
            pltpu.SemaphoreType.DMA((2,)),
            pltpu.SemaphoreType.DMA((2,)),
            pltpu.SemaphoreType.DMA((2,)),
            pltpu.SemaphoreType.DMA((2,)),
        ],
    )
    return f(node_features, coef, edge_src, edge_dst)


def _combine_body(p0_ref, p1_ref, o_ref):
    o_ref[...] = (p0_ref[0] + p0_ref[1]) + (p1_ref[0] + p1_ref[1])


def _combine(part0, part1):
    blk = 1000
    return pl.pallas_call(
        _combine_body,
        grid=(N // blk,),
        in_specs=[
            pl.BlockSpec((_NC, blk, D), lambda i: (0, i, 0)),
            pl.BlockSpec((_NC, blk, D), lambda i: (0, i, 0)),
        ],
        out_specs=pl.BlockSpec((blk, D), lambda i: (i, 0)),
        out_shape=jax.ShapeDtypeStruct((N, D), jnp.float32),
    )(part0, part1)


def kernel(node_features, edge_attr, edge_embedding, edge_index, W1, W2, W3):
    W1a = W1 * 0.25
    W2a = W2 * 0.125
    W3a = W3 * (0.125 / 32.0)

    embt = edge_embedding.T
    attrt = edge_attr.T
    parts = []
    for g in range(G):
        coef = _edge_coefficients(g, embt, attrt, W1a, W2a, W3a)
        src = lax.slice(edge_index[1], (g * EG,), ((g + 1) * EG,))
        dst = lax.slice(edge_index[0], (g * EG,), ((g + 1) * EG,))
        parts.append(_scatter_partials(node_features, coef, src, dst))
    return _combine(parts[0], parts[1])


# speedup vs baseline: 1.0470x; 1.0470x over previous
import jax
import jax.numpy as jnp
import numpy as np
from jax import lax
from jax.experimental import pallas as pl
from jax.experimental.pallas import tpu as pltpu
from jax.experimental.pallas import tpu_sc as plsc

N = 10000
E = 320000
D = 128
EMB = 16
H = 64

_z = np.linspace(-10.0, 10.0, 200001)
_pdf = np.exp(-0.5 * _z ** 2) / np.sqrt(2.0 * np.pi)
_a = np.logaddexp(0.0, _z) - np.log(2.0)
_SSP = float(1.0 / np.sqrt(np.trapz(_a ** 2 * _pdf, _z)))
_LOG2 = float(np.log(2.0))

G = 4
EG = E // G

_NC = 2
_NS = 16
_NW = _NC * _NS
CH = 80
NCHUNK = EG // CH
ITERS = (NCHUNK + _NW - 1) // _NW
N_PAD = 10240
ROWS_PER_SUB = N_PAD // _NS

BLK = 3200
BPG = EG // BLK


def _ssp(x):
    sp = jnp.maximum(x, 0.0) + jnp.log(1.0 + jnp.exp(-jnp.abs(x)))
    return (sp - _LOG2) * _SSP


def _coef_body(embt_ref, attr_ref, w1_ref, w2_ref, w3_ref, o_ref):
    h = _ssp(lax.dot_general(embt_ref[...], w1_ref[...],
                             (((0,), (0,)), ((), ())),
                             preferred_element_type=jnp.float32))
    h = _ssp(jnp.dot(h, w2_ref[...], preferred_element_type=jnp.float32))
    w = jnp.dot(h, w3_ref[...], preferred_element_type=jnp.float32)
    a = jnp.transpose(attr_ref[...])
    o_ref[...] = w * a


def _edge_coefficients(g, edge_embedding_t, edge_attr_t, W1a, W2a, W3a):
    return pl.pallas_call(
        _coef_body,
        grid=(BPG,),
        in_specs=[
            pl.BlockSpec((EMB, BLK), lambda i: (0, i + g * BPG)),
            pl.BlockSpec((1, BLK), lambda i: (0, i + g * BPG)),
            pl.BlockSpec((EMB, H), lambda i: (0, 0)),
            pl.BlockSpec((H, H), lambda i: (0, 0)),
            pl.BlockSpec((H, D), lambda i: (0, 0)),
        ],
        out_specs=pl.BlockSpec((BLK, D), lambda i: (i, 0)),
        out_shape=jax.ShapeDtypeStruct((EG, D), jnp.float32),
    )(edge_embedding_t, edge_attr_t, W1a, W2a, W3a)


def _sc_body(g, x_hbm, p_hbm, ei_hbm, out_hbm,
             src_v, dst_v, sdst_v, rows_v, p_v, acc_sh,
             s_src, s_dst, s_g, s_p):
    cid = lax.axis_index("c")
    sid = lax.axis_index("s")
    wid = sid * _NC + cid
    eoff = g * EG

    def _base(j):
        return (wid + j * _NW) * CH

    def _valid(j):
        return (wid + j * _NW) < NCHUNK

    def _issue_idx(j, b):
        pltpu.async_copy(ei_hbm.at[pl.ds(E + eoff + _base(j), CH)], src_v.at[b], s_src.at[b])
        pltpu.async_copy(ei_hbm.at[pl.ds(eoff + _base(j), CH)], dst_v.at[b], s_dst.at[b])

    def _wait_idx(j, b):
        pltpu.make_async_copy(ei_hbm.at[pl.ds(E + eoff + _base(j), CH)], src_v.at[b], s_src.at[b]).wait()
        pltpu.make_async_copy(ei_hbm.at[pl.ds(eoff + _base(j), CH)], dst_v.at[b], s_dst.at[b]).wait()

    def _issue_data(j, b):
        pltpu.async_copy(x_hbm.at[src_v.at[b]], rows_v.at[b], s_g.at[b])
        pltpu.async_copy(p_hbm.at[pl.ds(_base(j), CH)], p_v.at[b], s_p.at[b])

    def _wait_data(j, b):
        pltpu.make_async_copy(x_hbm.at[src_v.at[b]], rows_v.at[b], s_g.at[b]).wait()
        pltpu.make_async_copy(p_hbm.at[pl.ds(_base(j), CH)], p_v.at[b], s_p.at[b]).wait()

    def _zrow(i, carry):
        for k in range(D // 16):
            rows_v[0, i, pl.ds(k * 16, 16)] = jnp.zeros((16,), jnp.float32)
        return carry
    lax.fori_loop(0, CH, _zrow, 0)
    for t in range(ROWS_PER_SUB // CH):
        pltpu.sync_copy(rows_v.at[0],
                        acc_sh.at[pl.ds(sid * ROWS_PER_SUB + t * CH, CH)])
    plsc.subcore_barrier()

    pltpu.sync_copy(ei_hbm.at[pl.ds(E + eoff + _base(0), CH)], src_v.at[0])
    pltpu.sync_copy(ei_hbm.at[pl.ds(eoff + _base(0), CH)], dst_v.at[0])
    _issue_data(0, 0)
    _issue_idx(1, 1)

    def _outer(g, carry):
        for b in (0, 1):
            j = g * 2 + b
            b2 = 1 - b

            @pl.when(_valid(j + 1))
            def _():
                _wait_idx(j + 1, b2)
                _issue_data(j + 1, b2)

            @pl.when(_valid(j))
            def _():
                _wait_data(j, b)
                for k in range(CH // 16):
                    s = pl.ds(k * 16, 16)
                    sdst_v[b, s] = dst_v[b, s]

                @pl.when(_valid(j + 2))
                def _():
                    _issue_idx(j + 2, b)

                def _mul(i, c2):
                    for k in range(D // 16):
                        s = pl.ds(k * 16, 16)
                        p_v[b, i, s] = p_v[b, i, s] * rows_v[b, i, s]
                    return c2
                lax.fori_loop(0, CH, _mul, 0)
                pltpu.sync_copy(p_v.at[b], acc_sh.at[sdst_v.at[b]], add=True)
        return carry
    lax.fori_loop(0, (ITERS + 1) // 2, _outer, 0)
    plsc.subcore_barrier()

    r = sid * ROWS_PER_SUB
    pltpu.sync_copy(acc_sh.at[pl.ds(r, ROWS_PER_SUB)],
                    out_hbm.at[cid, pl.ds(r, ROWS_PER_SUB)])


def _scatter_partials(g, node_features, coef, edge_index_flat):
    mesh = plsc.VectorSubcoreMesh(core_axis_name="c", subcore_axis_name="s")

    def body(*args):
        _sc_body(g, *args)

    f = pl.kernel(
        body,
        out_type=jax.ShapeDtypeStruct((_NC, N_PAD, D), jnp.float32),
        mesh=mesh,
        scratch_types=[
            pltpu.VMEM((2, CH), jnp.int32),
            pltpu.VMEM((2, CH), jnp.int32),
            pltpu.VMEM((2, CH), jnp.int32),
            pltpu.VMEM((2, CH, D), jnp.float32),
            pltpu.VMEM((2, CH, D), jnp.float32),
            pltpu.VMEM_SHARED((N_PAD, D), jnp.float32),
            pltpu.SemaphoreType.DMA((2,)),
            pltpu.SemaphoreType.DMA((2,)),
            pltpu.SemaphoreType.DMA((2,)),
            pltpu.SemaphoreType.DMA((2,)),
        ],
    )
    return f(node_features, coef, edge_index_flat)


def _combine_body(p0_ref, p1_ref, p2_ref, p3_ref, o_ref):
    o_ref[...] = ((p0_ref[0] + p0_ref[1]) + (p1_ref[0] + p1_ref[1]) +
                  (p2_ref[0] + p2_ref[1]) + (p3_ref[0] + p3_ref[1]))


def _combine(parts):
    blk = 1000
    spec = pl.BlockSpec((_NC, blk, D), lambda i: (0, i, 0))
    return pl.pallas_call(
        _combine_body,
        grid=(N // blk,),
        in_specs=[spec] * G,
        out_specs=pl.BlockSpec((blk, D), lambda i: (i, 0)),
        out_shape=jax.ShapeDtypeStruct((N, D), jnp.float32),
    )(*parts)


def kernel(node_features, edge_attr, edge_embedding, edge_index, W1, W2, W3):
    W1a = W1 * 0.25
    W2a = W2 * 0.125
    W3a = W3 * (0.125 / 32.0)

    embt = edge_embedding.T
    attrt = edge_attr.T
    ei = edge_index.reshape(2 * E)
    parts = []
    for g in range(G):
        coef = _edge_coefficients(g, embt, attrt, W1a, W2a, W3a)
        parts.append(_scatter_partials(g, node_features, coef, ei))
    return _combine(parts)
